# baseline (speedup 1.0000x reference)
import functools

import jax
import jax.numpy as jnp
from jax import lax
from jax.experimental import pallas as pl
from jax.experimental.pallas import tpu as pltpu
from jax.experimental.pallas import tpu_sc as plsc

N = 10000
D = 128
E = 320000

NPAD = 10240
CHUNK = 128
CH_TOTAL = 2560
EPAD = CH_TOTAL * CHUNK
ROWS_PER_TILE = NPAD // 16

STAGES = ((128, 1), (256, 2), (512, 2), (768, 2), (896, 2))

BLK = 2048
EBLK = 4096


@functools.lru_cache(maxsize=None)
def _scatter_kernel():
    mesh = plsc.VectorSubcoreMesh(core_axis_name="c", subcore_axis_name="s",
                                  num_cores=1)
    nc = CH_TOTAL // 16
    nt = nc // 2

    @functools.partial(
        pl.kernel,
        out_type=jax.ShapeDtypeStruct((NPAD, D), jnp.float32),
        mesh=mesh,
        scratch_types=[
            pltpu.VMEM((CHUNK,), jnp.int32),
            pltpu.VMEM((CHUNK,), jnp.int32),
            pltpu.VMEM((CHUNK,), jnp.int32),
            pltpu.VMEM((CHUNK,), jnp.int32),
            pltpu.VMEM((CHUNK, D), jnp.float32),
            pltpu.VMEM((CHUNK, D), jnp.float32),
            pltpu.VMEM_SHARED((NPAD, D), jnp.float32),
            pltpu.SemaphoreType.DMA,
            pltpu.SemaphoreType.DMA,
            pltpu.SemaphoreType.DMA,
            pltpu.SemaphoreType.DMA,
            pltpu.SemaphoreType.DMA,
            pltpu.SemaphoreType.DMA,
        ],
    )
    def k(m_hbm, srcw_hbm, dstw_hbm, zeros_hbm, parts_hbm,
          idx_s0, idx_s1, idx_d0, idx_d1, rows0, rows1, agg,
          ss0, ss1, sd0, sd1, sg0, sg1):
        s = lax.axis_index("s")
        base = s * nc
        pltpu.sync_copy(zeros_hbm,
                        agg.at[pl.ds(s * ROWS_PER_TILE, ROWS_PER_TILE)])
        plsc.subcore_barrier()

        def start_idx(j, idx_s, idx_d, sem_s, sem_d):
            off = (base + j) * CHUNK
            pltpu.async_copy(srcw_hbm.at[pl.ds(off, CHUNK)], idx_s, sem_s)
            pltpu.async_copy(dstw_hbm.at[pl.ds(off, CHUNK)], idx_d, sem_d)

        def wait_idx(idx, sem):
            pltpu.make_async_copy(srcw_hbm.at[pl.ds(0, CHUNK)], idx, sem).wait()

        def start(idx, rows, sem):
            pltpu.async_copy(m_hbm.at[idx], rows, sem)

        def wait_gather(rows, sem):
            pltpu.make_async_copy(m_hbm.at[pl.ds(0, CHUNK)], rows, sem).wait()

        start_idx(0, idx_s0, idx_d0, ss0, sd0)
        wait_idx(idx_s0, ss0)
        start(idx_s0, rows0, sg0)
        start_idx(1, idx_s1, idx_d1, ss1, sd1)

        def body(t, carry):
            j0 = 2 * t
            wait_idx(idx_s1, ss1)
            start(idx_s1, rows1, sg1)
            wait_gather(rows0, sg0)
            wait_idx(idx_d0, sd0)
            pltpu.sync_copy(rows0, agg.at[idx_d0], add=True)

            @pl.when(t < nt - 1)
            def _():
                start_idx(j0 + 2, idx_s0, idx_d0, ss0, sd0)

            wait_gather(rows1, sg1)
            wait_idx(idx_d1, sd1)
            pltpu.sync_copy(rows1, agg.at[idx_d1], add=True)

            @pl.when(t < nt - 1)
            def _():
                wait_idx(idx_s0, ss0)
                start(idx_s0, rows0, sg0)
                start_idx(j0 + 3, idx_s1, idx_d1, ss1, sd1)

            return carry

        lax.fori_loop(0, nt, body, 0)
        plsc.subcore_barrier()
        pltpu.sync_copy(agg.at[pl.ds(s * ROWS_PER_TILE, ROWS_PER_TILE)],
                        parts_hbm.at[pl.ds(s * ROWS_PER_TILE, ROWS_PER_TILE)])

    return k


@functools.lru_cache(maxsize=None)
def _pair_kernel(nch, ncores):
    mesh = plsc.VectorSubcoreMesh(core_axis_name="c", subcore_axis_name="s",
                                  num_cores=ncores)
    nc = nch // (16 * ncores)
    nt = nc // 2
    nedge = nch * CHUNK

    @functools.partial(
        pl.kernel,
        out_type=(jax.ShapeDtypeStruct((nedge, D), jnp.float32),
                  jax.ShapeDtypeStruct((nedge, D), jnp.float32)),
        mesh=mesh,
        scratch_types=[
            pltpu.VMEM((nc * CHUNK,), jnp.int32),
            pltpu.VMEM((nc * CHUNK,), jnp.int32),
            pltpu.VMEM((CHUNK, D), jnp.float32),
            pltpu.VMEM((CHUNK, D), jnp.float32),
            pltpu.VMEM((CHUNK, D), jnp.float32),
            pltpu.VMEM((CHUNK, D), jnp.float32),
            pltpu.SemaphoreType.DMA,
            pltpu.SemaphoreType.DMA,
            pltpu.SemaphoreType.DMA,
            pltpu.SemaphoreType.DMA,
            pltpu.SemaphoreType.DMA,
            pltpu.SemaphoreType.DMA,
            pltpu.SemaphoreType.DMA,
            pltpu.SemaphoreType.DMA,
        ],
    )
    def k(h_hbm, srcw_hbm, dstw_hbm, hu_hbm, hv_hbm,
          idx_u, idx_v, ru0, ru1, rv0, rv1,
          gu0, gu1, gv0, gv1, wu0, wu1, wv0, wv1):
        c = lax.axis_index("c")
        s = lax.axis_index("s")
        wid = c * 16 + s
        base = wid * nc
        pltpu.sync_copy(srcw_hbm.at[pl.ds(base * CHUNK, nc * CHUNK)], idx_u)
        pltpu.sync_copy(dstw_hbm.at[pl.ds(base * CHUNK, nc * CHUNK)], idx_v)

        def start(j, ru, rv, sgu, sgv):
            pltpu.async_copy(h_hbm.at[idx_u.at[pl.ds(j * CHUNK, CHUNK)]], ru, sgu)
            pltpu.async_copy(h_hbm.at[idx_v.at[pl.ds(j * CHUNK, CHUNK)]], rv, sgv)

        def wait_gather(rows, sem):
            pltpu.make_async_copy(h_hbm.at[pl.ds(0, CHUNK)], rows, sem).wait()

        def start_write(j, ru, rv, swu, swv):
            off = (base + j) * CHUNK
            pltpu.async_copy(ru, hu_hbm.at[pl.ds(off, CHUNK)], swu)
            pltpu.async_copy(rv, hv_hbm.at[pl.ds(off, CHUNK)], swv)

        def wait_write(rows, sem):
            pltpu.make_async_copy(rows, hu_hbm.at[pl.ds(0, CHUNK)], sem).wait()

        start(0, ru0, rv0, gu0, gv0)

        def body(t, carry):
            j0 = 2 * t

            @pl.when(t > 0)
            def _():
                wait_write(ru1, wu1)
                wait_write(rv1, wv1)

            start(j0 + 1, ru1, rv1, gu1, gv1)
            wait_gather(ru0, gu0)
            wait_gather(rv0, gv0)
            start_write(j0, ru0, rv0, wu0, wv0)

            @pl.when(t < nt - 1)
            def _():
                wait_write(ru0, wu0)
                wait_write(rv0, wv0)
                start(j0 + 2, ru0, rv0, gu0, gv0)

            wait_gather(ru1, gu1)
            wait_gather(rv1, gv1)
            start_write(j0 + 1, ru1, rv1, wu1, wv1)
            return carry

        lax.fori_loop(0, nt, body, 0)
        wait_write(ru0, wu0)
        wait_write(rv0, wv0)
        wait_write(ru1, wu1)
        wait_write(rv1, wv1)

    return k


def _msg_body(h_ref, wt_ref, b_ref, o_ref):
    i = pl.program_id(0)
    y = jnp.dot(h_ref[...], wt_ref[...], preferred_element_type=jnp.float32)
    y = jnp.maximum(y + b_ref[...], 0.0)
    rows = lax.broadcasted_iota(jnp.int32, y.shape, 0) + i * BLK
    o_ref[...] = jnp.where(rows < N, y, 0.0)


def _msg(h_pad, WmT, bm):
    return pl.pallas_call(
        _msg_body,
        grid=(NPAD // BLK,),
        in_specs=[pl.BlockSpec((BLK, D), lambda i: (i, 0)),
                  pl.BlockSpec((D, D), lambda i: (0, 0)),
                  pl.BlockSpec((1, D), lambda i: (0, 0))],
        out_specs=pl.BlockSpec((BLK, D), lambda i: (i, 0)),
        out_shape=jax.ShapeDtypeStruct((NPAD, D), jnp.float32),
    )(h_pad, WmT, bm)


def _gru_math(agg, hb, WuT, bu, WihT, bih, WhhT, bhh):
    msg = jnp.dot(agg, WuT, preferred_element_type=jnp.float32) + bu
    msg = jnp.maximum(msg, 0.0)
    gi = jnp.dot(msg, WihT, preferred_element_type=jnp.float32) + bih
    gh = jnp.dot(hb, WhhT, preferred_element_type=jnp.float32) + bhh
    r = jax.nn.sigmoid(gi[:, :D] + gh[:, :D])
    z = jax.nn.sigmoid(gi[:, D:2 * D] + gh[:, D:2 * D])
    n = jnp.tanh(gi[:, 2 * D:] + r * gh[:, 2 * D:])
    return (1.0 - z) * n + z * hb


def _upd_m_body(parts_ref, h_ref, WuT, bu, WihT, bih, WhhT, bhh, WmT, bm,
                h_out, m_out):
    i = pl.program_id(0)
    hn = _gru_math(parts_ref[...], h_ref[...], WuT[...], bu[...],
                   WihT[...], bih[...], WhhT[...], bhh[...])
    h_out[...] = hn
    y = jnp.dot(hn, WmT[...], preferred_element_type=jnp.float32)
    y = jnp.maximum(y + bm[...], 0.0)
    rows = lax.broadcasted_iota(jnp.int32, y.shape, 0) + i * BLK
    m_out[...] = jnp.where(rows < N, y, 0.0)


def _upd_m(parts, h_pad, WuT, bu, WihT, bih, WhhT, bhh, WmT, bm):
    full = lambda shape: pl.BlockSpec(shape, lambda i: tuple(0 for _ in shape))
    return pl.pallas_call(
        _upd_m_body,
        grid=(NPAD // BLK,),
        in_specs=[pl.BlockSpec((BLK, D), lambda i: (i, 0)),
                  pl.BlockSpec((BLK, D), lambda i: (i, 0)),
                  full((D, D)), full((1, D)),
                  full((D, 3 * D)), full((1, 3 * D)),
                  full((D, 3 * D)), full((1, 3 * D)),
                  full((D, D)), full((1, D))],
        out_specs=(pl.BlockSpec((BLK, D), lambda i: (i, 0)),
                   pl.BlockSpec((BLK, D), lambda i: (i, 0))),
        out_shape=(jax.ShapeDtypeStruct((NPAD, D), jnp.float32),
                   jax.ShapeDtypeStruct((NPAD, D), jnp.float32)),
    )(parts, h_pad, WuT, bu, WihT, bih, WhhT, bhh, WmT, bm)


def _upd_body(parts_ref, h_ref, WuT, bu, WihT, bih, WhhT, bhh, h_out):
    h_out[...] = _gru_math(parts_ref[...], h_ref[...], WuT[...],
                           bu[...], WihT[...], bih[...], WhhT[...], bhh[...])


def _upd(parts, h_pad, WuT, bu, WihT, bih, WhhT, bhh):
    full = lambda shape: pl.BlockSpec(shape, lambda i: tuple(0 for _ in shape))
    return pl.pallas_call(
        _upd_body,
        grid=(NPAD // BLK,),
        in_specs=[pl.BlockSpec((BLK, D), lambda i: (i, 0)),
                  pl.BlockSpec((BLK, D), lambda i: (i, 0)),
                  full((D, D)), full((1, D)),
                  full((D, 3 * D)), full((1, 3 * D)),
                  full((D, 3 * D)), full((1, 3 * D))],
        out_specs=pl.BlockSpec((BLK, D), lambda i: (i, 0)),
        out_shape=jax.ShapeDtypeStruct((NPAD, D), jnp.float32),
    )(parts, h_pad, WuT, bu, WihT, bih, WhhT, bhh)


def _score_body(hu_ref, hv_ref, W1T_ref, b1_ref, w2_ref, b2_ref, o_ref):
    u = hu_ref[...]
    v = hv_ref[...]
    phi = jnp.concatenate([
        u.astype(jnp.bfloat16),
        v.astype(jnp.bfloat16),
        jnp.abs(u - v).astype(jnp.bfloat16),
        (u * v).astype(jnp.bfloat16),
    ], axis=1)
    hid = jnp.dot(phi, W1T_ref[...], preferred_element_type=jnp.float32)
    hid = jnp.maximum(hid + b1_ref[...], 0.0)
    o_ref[...] = jnp.sum(hid * w2_ref[...] + b2_ref[...], axis=1)


@functools.lru_cache(maxsize=None)
def _score_call(nedge):
    full = lambda shape: pl.BlockSpec(shape, lambda i: tuple(0 for _ in shape))
    return pl.pallas_call(
        _score_body,
        grid=(nedge // EBLK,),
        in_specs=[pl.BlockSpec((EBLK, D), lambda i: (i, 0)),
                  pl.BlockSpec((EBLK, D), lambda i: (i, 0)),
                  full((4 * D, D)), full((1, D)), full((1, D)), full((1, D))],
        out_specs=pl.BlockSpec((EBLK,), lambda i: (i,)),
        out_shape=jax.ShapeDtypeStruct((nedge,), jnp.float32),
    )


def kernel(h, edge_index, Wm0, bm0, Wm1, bm1, Wu0, bu0, Wu1, bu1,
           W_ih, b_ih, W_hh, b_hh, We1, be1, We2, be2):
    src = edge_index[0]
    dst = edge_index[1]
    padi = jnp.full((EPAD - E,), N, jnp.int32)
    srcf = jnp.concatenate([src, padi])
    dstf = jnp.concatenate([dst, padi])
    h0 = jnp.pad(h, ((0, NPAD - N), (0, 0)))
    zrows = jnp.zeros((ROWS_PER_TILE, D), jnp.float32)

    scatter = _scatter_kernel()

    m0 = _msg(h0, Wm0.T, bm0[None])
    parts0 = scatter(m0, srcf, dstf, zrows)
    h1, m1 = _upd_m(parts0, h0, Wu0.T, bu0[None], W_ih.T, b_ih[None],
                    W_hh.T, b_hh[None], Wm1.T, bm1[None])
    parts1 = scatter(m1, srcf, dstf, zrows)
    h2 = _upd(parts1, h1, Wu1.T, bu1[None], W_ih.T, b_ih[None],
              W_hh.T, b_hh[None])
    b2row = jnp.full((1, D), be2[0] / D, jnp.float32)
    W1T = We1.T.astype(jnp.bfloat16)
    b1 = be1[None]
    scs = []
    off = 0
    for nch, ncores in STAGES:
        o = off * CHUNK
        n_ = nch * CHUNK
        hu, hv = _pair_kernel(nch, ncores)(h2, srcf[o:o + n_], dstf[o:o + n_])
        scs.append(_score_call(n_)(hu, hv, W1T, b1, We2, b2row))
        off += nch
    return jnp.concatenate(scs)[:E]


# speedup vs baseline: 1.2101x; 1.2101x over previous
import functools

import jax
import jax.numpy as jnp
from jax import lax
from jax.experimental import pallas as pl
from jax.experimental.pallas import tpu as pltpu
from jax.experimental.pallas import tpu_sc as plsc

N = 10000
D = 128
E = 320000

NPAD = 10240
CHUNK = 128
CH_TOTAL = 2560
EPAD = CH_TOTAL * CHUNK
ROWS_PER_TILE = NPAD // 16

STAGES = ((128, 1), (256, 1), (512, 1), (768, 1), (896, 1))

BLK = 2048
EBLK = 4096


A_SC, B_SC = 120, 40


@functools.lru_cache(maxsize=None)
def _scatter_kernel():
    mesh = plsc.VectorSubcoreMesh(core_axis_name="c", subcore_axis_name="s",
                                  num_cores=2)

    @functools.partial(
        pl.kernel,
        out_type=jax.ShapeDtypeStruct((2, NPAD, D), jnp.float32),
        mesh=mesh,
        scratch_types=[
            pltpu.VMEM((CHUNK,), jnp.int32),
            pltpu.VMEM((CHUNK,), jnp.int32),
            pltpu.VMEM((CHUNK,), jnp.int32),
            pltpu.VMEM((CHUNK,), jnp.int32),
            pltpu.VMEM((CHUNK, D), jnp.float32),
            pltpu.VMEM((CHUNK, D), jnp.float32),
            pltpu.VMEM_SHARED((NPAD, D), jnp.float32),
            pltpu.SemaphoreType.DMA,
            pltpu.SemaphoreType.DMA,
            pltpu.SemaphoreType.DMA,
            pltpu.SemaphoreType.DMA,
            pltpu.SemaphoreType.DMA,
            pltpu.SemaphoreType.DMA,
        ],
    )
    def k(m_hbm, srcw_hbm, dstw_hbm, zeros_hbm, parts_hbm,
          idx_s0, idx_s1, idx_d0, idx_d1, rows0, rows1, agg,
          ss0, ss1, sd0, sd1, sg0, sg1):
        c = lax.axis_index("c")
        s = lax.axis_index("s")
        nc = A_SC - c * (A_SC - B_SC)
        base = c * 16 * A_SC + s * nc
        nt = nc // 2
        pltpu.sync_copy(zeros_hbm,
                        agg.at[pl.ds(s * ROWS_PER_TILE, ROWS_PER_TILE)])
        plsc.subcore_barrier()

        def start_idx(j, idx_s, idx_d, sem_s, sem_d):
            off = (base + j) * CHUNK
            pltpu.async_copy(srcw_hbm.at[pl.ds(off, CHUNK)], idx_s, sem_s)
            pltpu.async_copy(dstw_hbm.at[pl.ds(off, CHUNK)], idx_d, sem_d)

        def wait_idx(idx, sem):
            pltpu.make_async_copy(srcw_hbm.at[pl.ds(0, CHUNK)], idx, sem).wait()

        def start(idx, rows, sem):
            pltpu.async_copy(m_hbm.at[idx], rows, sem)

        def wait_gather(rows, sem):
            pltpu.make_async_copy(m_hbm.at[pl.ds(0, CHUNK)], rows, sem).wait()

        start_idx(0, idx_s0, idx_d0, ss0, sd0)
        wait_idx(idx_s0, ss0)
        start(idx_s0, rows0, sg0)
        start_idx(1, idx_s1, idx_d1, ss1, sd1)

        def body(t, carry):
            j0 = 2 * t
            wait_idx(idx_s1, ss1)
            start(idx_s1, rows1, sg1)
            wait_gather(rows0, sg0)
            wait_idx(idx_d0, sd0)
            pltpu.sync_copy(rows0, agg.at[idx_d0], add=True)

            @pl.when(t < nt - 1)
            def _():
                start_idx(j0 + 2, idx_s0, idx_d0, ss0, sd0)

            wait_gather(rows1, sg1)
            wait_idx(idx_d1, sd1)
            pltpu.sync_copy(rows1, agg.at[idx_d1], add=True)

            @pl.when(t < nt - 1)
            def _():
                wait_idx(idx_s0, ss0)
                start(idx_s0, rows0, sg0)
                start_idx(j0 + 3, idx_s1, idx_d1, ss1, sd1)

            return carry

        lax.fori_loop(0, nt, body, 0)
        plsc.subcore_barrier()
        pltpu.sync_copy(agg.at[pl.ds(s * ROWS_PER_TILE, ROWS_PER_TILE)],
                        parts_hbm.at[c, pl.ds(s * ROWS_PER_TILE, ROWS_PER_TILE)])

    return k


@functools.lru_cache(maxsize=None)
def _pair_kernel(nch, ncores):
    mesh = plsc.VectorSubcoreMesh(core_axis_name="c", subcore_axis_name="s",
                                  num_cores=ncores)
    nc = nch // (16 * ncores)
    nt = nc // 2
    nedge = nch * CHUNK

    @functools.partial(
        pl.kernel,
        out_type=(jax.ShapeDtypeStruct((nedge, D), jnp.float32),
                  jax.ShapeDtypeStruct((nedge, D), jnp.float32)),
        mesh=mesh,
        scratch_types=[
            pltpu.VMEM((nc * CHUNK,), jnp.int32),
            pltpu.VMEM((nc * CHUNK,), jnp.int32),
            pltpu.VMEM((CHUNK, D), jnp.float32),
            pltpu.VMEM((CHUNK, D), jnp.float32),
            pltpu.VMEM((CHUNK, D), jnp.float32),
            pltpu.VMEM((CHUNK, D), jnp.float32),
            pltpu.SemaphoreType.DMA,
            pltpu.SemaphoreType.DMA,
            pltpu.SemaphoreType.DMA,
            pltpu.SemaphoreType.DMA,
            pltpu.SemaphoreType.DMA,
            pltpu.SemaphoreType.DMA,
            pltpu.SemaphoreType.DMA,
            pltpu.SemaphoreType.DMA,
        ],
    )
    def k(h_hbm, srcw_hbm, dstw_hbm, hu_hbm, hv_hbm,
          idx_u, idx_v, ru0, ru1, rv0, rv1,
          gu0, gu1, gv0, gv1, wu0, wu1, wv0, wv1):
        c = lax.axis_index("c")
        s = lax.axis_index("s")
        wid = c * 16 + s
        base = wid * nc
        pltpu.sync_copy(srcw_hbm.at[pl.ds(base * CHUNK, nc * CHUNK)], idx_u)
        pltpu.sync_copy(dstw_hbm.at[pl.ds(base * CHUNK, nc * CHUNK)], idx_v)

        def start(j, ru, rv, sgu, sgv):
            pltpu.async_copy(h_hbm.at[idx_u.at[pl.ds(j * CHUNK, CHUNK)]], ru, sgu)
            pltpu.async_copy(h_hbm.at[idx_v.at[pl.ds(j * CHUNK, CHUNK)]], rv, sgv)

        def wait_gather(rows, sem):
            pltpu.make_async_copy(h_hbm.at[pl.ds(0, CHUNK)], rows, sem).wait()

        def start_write(j, ru, rv, swu, swv):
            off = (base + j) * CHUNK
            pltpu.async_copy(ru, hu_hbm.at[pl.ds(off, CHUNK)], swu)
            pltpu.async_copy(rv, hv_hbm.at[pl.ds(off, CHUNK)], swv)

        def wait_write(rows, sem):
            pltpu.make_async_copy(rows, hu_hbm.at[pl.ds(0, CHUNK)], sem).wait()

        start(0, ru0, rv0, gu0, gv0)

        def body(t, carry):
            j0 = 2 * t

            @pl.when(t > 0)
            def _():
                wait_write(ru1, wu1)
                wait_write(rv1, wv1)

            start(j0 + 1, ru1, rv1, gu1, gv1)
            wait_gather(ru0, gu0)
            wait_gather(rv0, gv0)
            start_write(j0, ru0, rv0, wu0, wv0)

            @pl.when(t < nt - 1)
            def _():
                wait_write(ru0, wu0)
                wait_write(rv0, wv0)
                start(j0 + 2, ru0, rv0, gu0, gv0)

            wait_gather(ru1, gu1)
            wait_gather(rv1, gv1)
            start_write(j0 + 1, ru1, rv1, wu1, wv1)
            return carry

        lax.fori_loop(0, nt, body, 0)
        wait_write(ru0, wu0)
        wait_write(rv0, wv0)
        wait_write(ru1, wu1)
        wait_write(rv1, wv1)

    return k


def _msg_body(h_ref, wt_ref, b_ref, o_ref):
    i = pl.program_id(0)
    y = jnp.dot(h_ref[...], wt_ref[...], preferred_element_type=jnp.float32)
    y = jnp.maximum(y + b_ref[...], 0.0)
    rows = lax.broadcasted_iota(jnp.int32, y.shape, 0) + i * BLK
    o_ref[...] = jnp.where(rows < N, y, 0.0)


def _msg(h_pad, WmT, bm):
    return pl.pallas_call(
        _msg_body,
        grid=(NPAD // BLK,),
        in_specs=[pl.BlockSpec((BLK, D), lambda i: (i, 0)),
                  pl.BlockSpec((D, D), lambda i: (0, 0)),
                  pl.BlockSpec((1, D), lambda i: (0, 0))],
        out_specs=pl.BlockSpec((BLK, D), lambda i: (i, 0)),
        out_shape=jax.ShapeDtypeStruct((NPAD, D), jnp.float32),
    )(h_pad, WmT, bm)


def _gru_math(agg, hb, WuT, bu, WihT, bih, WhhT, bhh):
    msg = jnp.dot(agg, WuT, preferred_element_type=jnp.float32) + bu
    msg = jnp.maximum(msg, 0.0)
    gi = jnp.dot(msg, WihT, preferred_element_type=jnp.float32) + bih
    gh = jnp.dot(hb, WhhT, preferred_element_type=jnp.float32) + bhh
    r = jax.nn.sigmoid(gi[:, :D] + gh[:, :D])
    z = jax.nn.sigmoid(gi[:, D:2 * D] + gh[:, D:2 * D])
    n = jnp.tanh(gi[:, 2 * D:] + r * gh[:, 2 * D:])
    return (1.0 - z) * n + z * hb


def _upd_m_body(parts_ref, h_ref, WuT, bu, WihT, bih, WhhT, bhh, WmT, bm,
                h_out, m_out):
    i = pl.program_id(0)
    hn = _gru_math(parts_ref[0] + parts_ref[1], h_ref[...], WuT[...], bu[...],
                   WihT[...], bih[...], WhhT[...], bhh[...])
    h_out[...] = hn
    y = jnp.dot(hn, WmT[...], preferred_element_type=jnp.float32)
    y = jnp.maximum(y + bm[...], 0.0)
    rows = lax.broadcasted_iota(jnp.int32, y.shape, 0) + i * BLK
    m_out[...] = jnp.where(rows < N, y, 0.0)


def _upd_m(parts, h_pad, WuT, bu, WihT, bih, WhhT, bhh, WmT, bm):
    full = lambda shape: pl.BlockSpec(shape, lambda i: tuple(0 for _ in shape))
    return pl.pallas_call(
        _upd_m_body,
        grid=(NPAD // BLK,),
        in_specs=[pl.BlockSpec((2, BLK, D), lambda i: (0, i, 0)),
                  pl.BlockSpec((BLK, D), lambda i: (i, 0)),
                  full((D, D)), full((1, D)),
                  full((D, 3 * D)), full((1, 3 * D)),
                  full((D, 3 * D)), full((1, 3 * D)),
                  full((D, D)), full((1, D))],
        out_specs=(pl.BlockSpec((BLK, D), lambda i: (i, 0)),
                   pl.BlockSpec((BLK, D), lambda i: (i, 0))),
        out_shape=(jax.ShapeDtypeStruct((NPAD, D), jnp.float32),
                   jax.ShapeDtypeStruct((NPAD, D), jnp.float32)),
    )(parts, h_pad, WuT, bu, WihT, bih, WhhT, bhh, WmT, bm)


def _upd_body(parts_ref, h_ref, WuT, bu, WihT, bih, WhhT, bhh, h_out):
    h_out[...] = _gru_math(parts_ref[0] + parts_ref[1], h_ref[...], WuT[...],
                           bu[...], WihT[...], bih[...], WhhT[...], bhh[...])


def _upd(parts, h_pad, WuT, bu, WihT, bih, WhhT, bhh):
    full = lambda shape: pl.BlockSpec(shape, lambda i: tuple(0 for _ in shape))
    return pl.pallas_call(
        _upd_body,
        grid=(NPAD // BLK,),
        in_specs=[pl.BlockSpec((2, BLK, D), lambda i: (0, i, 0)),
                  pl.BlockSpec((BLK, D), lambda i: (i, 0)),
                  full((D, D)), full((1, D)),
                  full((D, 3 * D)), full((1, 3 * D)),
                  full((D, 3 * D)), full((1, 3 * D))],
        out_specs=pl.BlockSpec((BLK, D), lambda i: (i, 0)),
        out_shape=jax.ShapeDtypeStruct((NPAD, D), jnp.float32),
    )(parts, h_pad, WuT, bu, WihT, bih, WhhT, bhh)


def _score_body(hu_ref, hv_ref, W1T_ref, b1_ref, w2_ref, b2_ref, o_ref):
    u = hu_ref[...]
    v = hv_ref[...]
    phi = jnp.concatenate([
        u.astype(jnp.bfloat16),
        v.astype(jnp.bfloat16),
        jnp.abs(u - v).astype(jnp.bfloat16),
        (u * v).astype(jnp.bfloat16),
    ], axis=1)
    hid = jnp.dot(phi, W1T_ref[...], preferred_element_type=jnp.float32)
    hid = jnp.maximum(hid + b1_ref[...], 0.0)
    o_ref[...] = jnp.sum(hid * w2_ref[...] + b2_ref[...], axis=1)


@functools.lru_cache(maxsize=None)
def _score_call(nedge):
    full = lambda shape: pl.BlockSpec(shape, lambda i: tuple(0 for _ in shape))
    return pl.pallas_call(
        _score_body,
        grid=(nedge // EBLK,),
        in_specs=[pl.BlockSpec((EBLK, D), lambda i: (i, 0)),
                  pl.BlockSpec((EBLK, D), lambda i: (i, 0)),
                  full((4 * D, D)), full((1, D)), full((1, D)), full((1, D))],
        out_specs=pl.BlockSpec((EBLK,), lambda i: (i,)),
        out_shape=jax.ShapeDtypeStruct((nedge,), jnp.float32),
    )


def kernel(h, edge_index, Wm0, bm0, Wm1, bm1, Wu0, bu0, Wu1, bu1,
           W_ih, b_ih, W_hh, b_hh, We1, be1, We2, be2):
    src = edge_index[0]
    dst = edge_index[1]
    padi = jnp.full((EPAD - E,), N, jnp.int32)
    srcf = jnp.concatenate([src, padi])
    dstf = jnp.concatenate([dst, padi])
    h0 = jnp.pad(h, ((0, NPAD - N), (0, 0)))
    zrows = jnp.zeros((ROWS_PER_TILE, D), jnp.float32)

    scatter = _scatter_kernel()

    m0 = _msg(h0, Wm0.T, bm0[None])
    parts0 = scatter(m0, srcf, dstf, zrows)
    h1, m1 = _upd_m(parts0, h0, Wu0.T, bu0[None], W_ih.T, b_ih[None],
                    W_hh.T, b_hh[None], Wm1.T, bm1[None])
    parts1 = scatter(m1, srcf, dstf, zrows)
    h2 = _upd(parts1, h1, Wu1.T, bu1[None], W_ih.T, b_ih[None],
              W_hh.T, b_hh[None])
    b2row = jnp.full((1, D), be2[0] / D, jnp.float32)
    W1T = We1.T.astype(jnp.bfloat16)
    b1 = be1[None]
    scs = []
    off = 0
    for nch, ncores in STAGES:
        o = off * CHUNK
        n_ = nch * CHUNK
        hu, hv = _pair_kernel(nch, ncores)(h2, srcf[o:o + n_], dstf[o:o + n_])
        scs.append(_score_call(n_)(hu, hv, W1T, b1, We2, b2row))
        off += nch
    return jnp.concatenate(scs)[:E]
